# baseline (speedup 1.0000x reference)
import functools

import jax
import jax.numpy as jnp
from jax import lax
from jax.experimental import pallas as pl
from jax.experimental.pallas import tpu as pltpu
from jax.experimental.pallas import tpu_sc as plsc

_BATCH_SIZE = 4096
_NUM_DATA = 65536
_D = 256
_NUM_BATCHES = (_NUM_DATA + _BATCH_SIZE - 1) // _BATCH_SIZE

_NC = 2
_NS = 16
_NW = _NC * _NS
_B_PER_W = _BATCH_SIZE // _NW

_mesh = plsc.VectorSubcoreMesh(
    core_axis_name="c", subcore_axis_name="s", num_cores=_NC, num_subcores=_NS
)


@functools.partial(
    pl.kernel,
    mesh=_mesh,
    out_type=jax.ShapeDtypeStruct((_BATCH_SIZE, _D), jnp.float32),
    scratch_types=[
        pltpu.VMEM((_B_PER_W,), jnp.int32),
        pltpu.VMEM((_B_PER_W, _D), jnp.float32),
        pltpu.SemaphoreType.DMA,
    ],
)
def _gather_rows(table_hbm, idx_hbm, out_hbm, idx_v, rows_v, sem):
    wid = lax.axis_index("s") * _NC + lax.axis_index("c")
    base = wid * _B_PER_W
    pltpu.sync_copy(idx_hbm.at[pl.ds(base, _B_PER_W)], idx_v)
    pltpu.async_copy(table_hbm.at[idx_v], rows_v, sem).wait()
    pltpu.sync_copy(rows_v, out_hbm.at[pl.ds(base, _B_PER_W)])


def kernel(data, inds, cursor):
    cursor = jnp.asarray(cursor, jnp.int32)
    start = cursor * _BATCH_SIZE
    data_is = lax.dynamic_slice_in_dim(inds, start, _BATCH_SIZE, axis=0)

    indexed_data = _gather_rows(data, data_is)

    last_batch = jnp.equal(cursor, _NUM_BATCHES - 1)
    batch_remainder = _NUM_DATA % _BATCH_SIZE
    mask = jnp.where(
        jnp.logical_and(last_batch, batch_remainder > 0),
        jnp.arange(_BATCH_SIZE, dtype=jnp.int32) < batch_remainder,
        jnp.ones(_BATCH_SIZE, dtype=jnp.int32),
    )
    return (indexed_data, mask, last_batch)
